# baseline (speedup 1.0000x reference)
import dataclasses
import functools

import jax
import jax.numpy as jnp
from jax import lax
from jax.experimental import pallas as pl
from jax.experimental.pallas import tpu as pltpu
from jax.experimental.pallas import tpu_sc as plsc

N = 10000
NP = 10240
D = 256
H = 8
G = 16
E = 160000
EP = 163840
DH = D // H
NC = 2
NT = 16
HPC = H // NC
GW = HPC * DH
EPT = EP // NT
C = 32
C2 = 512
NCHUNK = EPT // C
RPT = NP // NT
ZROWS = 16
SPN = HPC * NP
BR = 1024
NW = NC * NT
INV_SQRT_DH = 1.0 / (DH ** 0.5)


def _sel_mat():
    r = lax.broadcasted_iota(jnp.int32, (H, 4 * NW), 1)
    h = lax.broadcasted_iota(jnp.int32, (H, 4 * NW), 0)
    head = (r // (4 * NT)) * HPC + (r % HPC)
    return (head == h).astype(jnp.float32)


def _expand8_mat():
    r = lax.broadcasted_iota(jnp.int32, (H, D), 0)
    c = lax.broadcasted_iota(jnp.int32, (H, D), 1)
    return (r == c // DH).astype(jnp.float32)


def _agg_from_acc(acc_block, sp_block):
    s_heads = jnp.dot(_sel_mat(), sp_block,
                      preferred_element_type=jnp.float32)
    recip = 1.0 / (s_heads + 1e-16)
    dn = (((0,), (0,)), ((), ()))
    r_cols = lax.dot_general(recip, _expand8_mat(), dn,
                             preferred_element_type=jnp.float32)
    agg = jnp.concatenate([acc_block[0], acc_block[1]], axis=1)
    return agg * r_cols


def _proj_stores(h, wq_ref, wk_ref, wv_ref, ws_ref, q_ref, k_ref, v_ref, s_ref):
    for w_ref, o_ref in ((wq_ref, q_ref), (wk_ref, k_ref), (wv_ref, v_ref)):
        r = jnp.dot(h, w_ref[...], preferred_element_type=jnp.float32)
        o_ref[0] = r[:, :GW]
        o_ref[1] = r[:, GW:]
    s_ref[...] = jnp.dot(h, ws_ref[...], preferred_element_type=jnp.float32)


def _proj_body(x_ref, wq_ref, wk_ref, wv_ref, ws_ref, q_ref, k_ref, v_ref, s_ref):
    _proj_stores(x_ref[...], wq_ref, wk_ref, wv_ref, ws_ref,
                 q_ref, k_ref, v_ref, s_ref)


def _mid_body(acc_ref, sp_ref, skip_ref, wq_ref, wk_ref, wv_ref, ws_ref,
              q_ref, k_ref, v_ref, s_ref):
    h1 = _agg_from_acc(acc_ref[...], sp_ref[...]) + skip_ref[...]
    _proj_stores(h1, wq_ref, wk_ref, wv_ref, ws_ref, q_ref, k_ref, v_ref, s_ref)


def _project(x, Wq, Wk, Wv, Ws):
    wspec = pl.BlockSpec((D, D), lambda i: (0, 0))
    return pl.pallas_call(
        _proj_body,
        grid=(NP // BR,),
        in_specs=[pl.BlockSpec((BR, D), lambda i: (i, 0))] + [wspec] * 4,
        out_specs=[pl.BlockSpec((NC, BR, GW), lambda i: (0, i, 0))] * 3
        + [pl.BlockSpec((BR, D), lambda i: (i, 0))],
        out_shape=[jax.ShapeDtypeStruct((NC, NP, GW), jnp.float32)] * 3
        + [jax.ShapeDtypeStruct((NP, D), jnp.float32)],
    )(x, Wq, Wk, Wv, Ws)


def _mid(acc, sp, skip, Wq, Wk, Wv, Ws):
    wspec = pl.BlockSpec((D, D), lambda i: (0, 0))
    return pl.pallas_call(
        _mid_body,
        grid=(NP // BR,),
        in_specs=[pl.BlockSpec((NC, BR, GW), lambda i: (0, i, 0)),
                  pl.BlockSpec((4 * NW, BR), lambda i: (0, i)),
                  pl.BlockSpec((BR, D), lambda i: (i, 0))] + [wspec] * 4,
        out_specs=[pl.BlockSpec((NC, BR, GW), lambda i: (0, i, 0))] * 3
        + [pl.BlockSpec((BR, D), lambda i: (i, 0))],
        out_shape=[jax.ShapeDtypeStruct((NC, NP, GW), jnp.float32)] * 3
        + [jax.ShapeDtypeStruct((NP, D), jnp.float32)],
    )(acc, sp, skip, Wq, Wk, Wv, Ws)


def _edge_sc(qt, kt, vt, src, dst):
    mesh = plsc.VectorSubcoreMesh(core_axis_name="c", subcore_axis_name="s")
    cp = pltpu.CompilerParams()
    if "needs_layout_passes" in pltpu.CompilerParams.__dataclass_fields__:
        cp = dataclasses.replace(cp, needs_layout_passes=False)

    @functools.partial(
        pl.kernel,
        out_type=[jax.ShapeDtypeStruct((NC * NP, GW), jnp.float32),
                  jax.ShapeDtypeStruct((NC * EP, 16), jnp.float32)],
        mesh=mesh,
        compiler_params=cp,
        scratch_types=[
            pltpu.VMEM((C,), jnp.int32),
            pltpu.VMEM((C,), jnp.int32),
            pltpu.VMEM((C,), jnp.int32),
            pltpu.VMEM((C,), jnp.int32),
            pltpu.VMEM((C,), jnp.int32),
            pltpu.VMEM((C,), jnp.int32),
            pltpu.VMEM((C, GW), jnp.float32),
            pltpu.VMEM((C, GW), jnp.float32),
            pltpu.VMEM((C, GW), jnp.float32),
            pltpu.VMEM((C, GW), jnp.float32),
            pltpu.VMEM((C, GW), jnp.float32),
            pltpu.VMEM((C, GW), jnp.float32),
            pltpu.VMEM((C, 16), jnp.float32),
            pltpu.VMEM((C, 16), jnp.float32),
            pltpu.SemaphoreType.DMA,
            pltpu.SemaphoreType.DMA,
            pltpu.SemaphoreType.DMA,
            pltpu.SemaphoreType.DMA,
            pltpu.SemaphoreType.DMA,
            pltpu.SemaphoreType.DMA,
            pltpu.SemaphoreType.DMA,
            pltpu.SemaphoreType.DMA,
            pltpu.VMEM_SHARED((NP, GW), jnp.float32),
        ],
    )
    def k1(qt_h, kt_h, vt_h, srcg_h, dstg_h, zr_h, acc_h, w_h,
           sg0, sg1, dg0, dg1, dsc0, dsc1, qd0, qd1, kd0, kd1, vd0, vd1,
           wb0, wb1, isem0, isem1, gsem0, gsem1, ssem0, ssem1, wsem0, wsem1,
           accs):
        g = lax.axis_index("c")
        sid = lax.axis_index("s")
        goff = g * NP
        iota16 = lax.iota(jnp.int32, 16)
        ebase = sid * EPT

        sg = (sg0, sg1)
        dg = (dg0, dg1)
        dsc = (dsc0, dsc1)
        qd = (qd0, qd1)
        kd = (kd0, kd1)
        vd = (vd0, vd1)
        wb = (wb0, wb1)
        isem = (isem0, isem1)
        gsem = (gsem0, gsem1)
        ssem = (ssem0, ssem1)
        wsem = (wsem0, wsem1)

        pltpu.sync_copy(zr_h, accs.at[pl.ds(sid * RPT, RPT)])
        plsc.subcore_barrier()

        def idx_issue(j, s):
            base = g * EP + ebase + j * C
            pltpu.async_copy(srcg_h.at[pl.ds(base, C)], sg[s], isem[s])
            pltpu.async_copy(dstg_h.at[pl.ds(base, C)], dg[s], isem[s])

        def idx_wait(j, s):
            base = g * EP + ebase + j * C
            pltpu.make_async_copy(srcg_h.at[pl.ds(base, C)], sg[s],
                                  isem[s]).wait()
            pltpu.make_async_copy(dstg_h.at[pl.ds(base, C)], dg[s],
                                  isem[s]).wait()

        def gather_issue(s):
            pltpu.async_copy(qt_h.at[dg[s]], qd[s], gsem[s])
            pltpu.async_copy(kt_h.at[sg[s]], kd[s], gsem[s])
            pltpu.async_copy(vt_h.at[sg[s]], vd[s], gsem[s])

        def gather_wait(s):
            pltpu.make_async_copy(qt_h.at[dg[s]], qd[s], gsem[s]).wait()
            pltpu.make_async_copy(kt_h.at[sg[s]], kd[s], gsem[s]).wait()
            pltpu.make_async_copy(vt_h.at[sg[s]], vd[s], gsem[s]).wait()

        def out_issue(j, s):
            pltpu.async_copy(vd[s], accs.at[dsc[s]], ssem[s], add=True)
            pltpu.async_copy(wb[s], w_h.at[pl.ds(g * EP + ebase + j * C, C)],
                             wsem[s])

        def out_wait(j, s):
            pltpu.make_async_copy(vd[s], accs.at[dsc[s]], ssem[s]).wait()
            pltpu.make_async_copy(wb[s],
                                  w_h.at[pl.ds(g * EP + ebase + j * C, C)],
                                  wsem[s]).wait()

        def compute(s):
            for off in range(0, C, 16):
                sl = pl.ds(off, 16)
                dsc[s][sl] = dg[s][sl] - goff

            @pl.loop(0, C // 16)
            def _(grp):
                rows = iota16 + grp * 16
                for j in range(HPC):
                    cb = j * DH

                    a4 = [jnp.zeros((16,), jnp.float32) for _ in range(4)]
                    for dd in range(DH):
                        cols = cb + ((dd + iota16) & (DH - 1))
                        qv = plsc.load_gather(qd[s], [rows, cols])
                        kv = plsc.load_gather(kd[s], [rows, cols])
                        a4[dd % 4] = a4[dd % 4] + qv * kv
                    sc = (a4[0] + a4[1]) + (a4[2] + a4[3])
                    w = jnp.exp(sc * INV_SQRT_DH)
                    plsc.store_scatter(
                        wb[s], [rows, (rows + j) & 15], w)

                    for dd in range(DH):
                        cols = cb + ((dd + iota16) & (DH - 1))
                        vg = plsc.load_gather(vd[s], [rows, cols])
                        plsc.store_scatter(vd[s], [rows, cols], vg * w)

        idx_issue(0, 0)
        idx_issue(1, 1)
        idx_wait(0, 0)
        gather_issue(0)

        gather_wait(0)
        compute(0)
        idx_wait(1, 1)
        gather_issue(1)
        idx_issue(2, 0)
        out_issue(0, 0)

        @pl.loop(0, (NCHUNK - 2) // 2)
        def _(cp2):
            for half in range(2):
                j = 1 + cp2 * 2 + half
                s = (1 + half) % 2
                o = 1 - s
                gather_wait(s)
                compute(s)
                idx_wait(j + 1, o)
                out_wait(j - 1, o)
                gather_issue(o)

                @pl.when(j < NCHUNK - 2)
                def _(j=j, s=s):
                    idx_issue(j + 2, s)

                out_issue(j, s)

        gather_wait(1)
        compute(1)
        out_wait(NCHUNK - 2, 0)
        out_issue(NCHUNK - 1, 1)
        out_wait(NCHUNK - 1, 1)

        plsc.subcore_barrier()

        @pl.loop(0, RPT // ZROWS)
        def _(b):
            r0 = sid * RPT + b * ZROWS
            pltpu.sync_copy(accs.at[pl.ds(r0, ZROWS)],
                            acc_h.at[pl.ds(goff + r0, ZROWS)])

    @functools.partial(
        pl.kernel,
        out_type=jax.ShapeDtypeStruct((NW, SPN), jnp.float32),
        mesh=mesh,
        compiler_params=cp,
        scratch_types=[
            pltpu.VMEM((C2,), jnp.int32),
            pltpu.VMEM((C2, 16), jnp.float32),
            pltpu.VMEM((SPN,), jnp.float32),
            pltpu.SemaphoreType.DMA,
        ],
    )
    def k2(w_h, dst_h, sp_h, dbuf, wb, spart, sem):
        g = lax.axis_index("c")
        sid = lax.axis_index("s")
        zvec = jnp.zeros((16,), jnp.float32)
        iota16 = lax.iota(jnp.int32, 16)
        jmod = (iota16 % HPC) * NP
        jmask = iota16 < HPC

        @pl.loop(0, SPN // 16)
        def _(r):
            spart[pl.ds(r * 16, 16)] = zvec

        ebase = sid * EPT

        @pl.loop(0, EPT // C2)
        def _(ci):
            base = ebase + ci * C2
            cd = pltpu.async_copy(dst_h.at[pl.ds(base, C2)], dbuf, sem)
            cw = pltpu.async_copy(w_h.at[pl.ds(g * EP + base, C2)], wb, sem)
            cd.wait()
            cw.wait()

            def ebody(eg, carry):
                e0 = eg * 16
                dvec = dbuf[pl.ds(e0, 16)]
                for ei in range(16):
                    ee = e0 + ei
                    er = jnp.full((16,), ee, jnp.int32)
                    dsp = jnp.full((16,), dvec[ei], jnp.int32)
                    w4 = plsc.load_gather(wb, [er, (iota16 + ee) & 15])
                    plsc.addupdate_scatter(spart, [jmod + dsp], w4,
                                           mask=jmask)
                return carry

            lax.fori_loop(0, C2 // 16, ebody, 0)

        pltpu.sync_copy(spart, sp_h.at[g * NT + sid])

    pad = jnp.full((EP - E,), N, jnp.int32)
    srcp = jnp.concatenate([src, pad])
    dstp = jnp.concatenate([dst, pad])
    srcg = jnp.concatenate([srcp, srcp + NP])
    dstg = jnp.concatenate([dstp, dstp + NP])
    zr = jnp.zeros((RPT, GW), jnp.float32)
    acc, w_all = k1(qt, kt, vt, srcg, dstg, zr)
    sp = k2(w_all, dstp)
    return acc, sp


def _norm1_body(acc_ref, sp_ref, skip_ref, b_ref, h2_ref, s1_ref, s2_ref,
                cnt_ref):
    i = pl.program_id(0)
    h2 = _agg_from_acc(acc_ref[...], sp_ref[...]) + skip_ref[...]
    h2_ref[...] = h2
    onehot = (b_ref[...] == lax.broadcasted_iota(jnp.int32, (1, G), 1))
    onehot = onehot.astype(jnp.float32)
    dn = (((0,), (0,)), ((), ()))
    p1 = lax.dot_general(onehot, h2, dn, preferred_element_type=jnp.float32)
    p2 = lax.dot_general(onehot, h2 * h2, dn,
                         preferred_element_type=jnp.float32)
    pc = lax.dot_general(onehot, jnp.ones_like(h2), dn,
                         preferred_element_type=jnp.float32)

    @pl.when(i == 0)
    def _():
        s1_ref[...] = jnp.zeros_like(s1_ref)
        s2_ref[...] = jnp.zeros_like(s2_ref)
        cnt_ref[...] = jnp.zeros_like(cnt_ref)

    s1_ref[...] += p1
    s2_ref[...] += p2
    cnt_ref[...] += pc


def _norm1(acc, sp, skip, batch2d):
    return pl.pallas_call(
        _norm1_body,
        grid=(NP // BR,),
        in_specs=[pl.BlockSpec((NC, BR, GW), lambda i: (0, i, 0)),
                  pl.BlockSpec((4 * NW, BR), lambda i: (0, i)),
                  pl.BlockSpec((BR, D), lambda i: (i, 0)),
                  pl.BlockSpec((BR, 1), lambda i: (i, 0))],
        out_specs=[pl.BlockSpec((BR, D), lambda i: (i, 0)),
                   pl.BlockSpec((G, D), lambda i: (0, 0)),
                   pl.BlockSpec((G, D), lambda i: (0, 0)),
                   pl.BlockSpec((G, D), lambda i: (0, 0))],
        out_shape=[jax.ShapeDtypeStruct((NP, D), jnp.float32),
                   jax.ShapeDtypeStruct((G, D), jnp.float32),
                   jax.ShapeDtypeStruct((G, D), jnp.float32),
                   jax.ShapeDtypeStruct((G, D), jnp.float32)],
    )(acc, sp, skip, batch2d)


def _norm2_body(h2_ref, b_ref, s1_ref, s2_ref, cnt_ref,
                gamma_ref, beta_ref, ms_ref, o_ref):
    cnt = jnp.maximum(cnt_ref[...], 1.0)
    mean = s1_ref[...] / cnt
    ex2 = s2_ref[...] / cnt
    ms = ms_ref[...]
    var = ex2 - (2.0 * ms - ms * ms) * (mean * mean)
    onehot = (b_ref[...] == lax.broadcasted_iota(jnp.int32, (1, G), 1))
    onehot = onehot.astype(jnp.float32)
    mean_r = jnp.dot(onehot, mean, preferred_element_type=jnp.float32)
    var_r = jnp.dot(onehot, var, preferred_element_type=jnp.float32)
    xc = h2_ref[...] - ms * mean_r
    o_ref[...] = (gamma_ref[...] * xc * lax.rsqrt(var_r + 1e-5)
                  + beta_ref[...])


def _norm2(h2, batch2d, s1, s2, cnt, gamma, beta, ms):
    small = pl.BlockSpec((G, D), lambda i: (0, 0))
    vec = pl.BlockSpec((1, D), lambda i: (0, 0))
    return pl.pallas_call(
        _norm2_body,
        grid=(NP // BR,),
        in_specs=[pl.BlockSpec((BR, D), lambda i: (i, 0)),
                  pl.BlockSpec((BR, 1), lambda i: (i, 0)),
                  small, small, small, vec, vec, vec],
        out_specs=pl.BlockSpec((BR, D), lambda i: (i, 0)),
        out_shape=jax.ShapeDtypeStruct((NP, D), jnp.float32),
    )(h2, batch2d, s1, s2, cnt, gamma, beta, ms)


def kernel(x, edge_index, batch, Wq0, Wk0, Wv0, Ws0,
           Wq1, Wk1, Wv1, Ws1, gamma, beta, mean_scale):
    src = edge_index[0]
    dst = edge_index[1]
    xp = jnp.pad(x, ((0, NP - N), (0, 0)))
    batch2d = jnp.pad(batch, (0, NP - N), constant_values=G).reshape(NP, 1)

    q0, k0, v0, skip0 = _project(xp, Wq0, Wk0, Wv0, Ws0)
    acc0, sp0 = _edge_sc(q0.reshape(NC * NP, GW), k0.reshape(NC * NP, GW),
                         v0.reshape(NC * NP, GW), src, dst)
    q1, k1, v1, skip1 = _mid(acc0.reshape(NC, NP, GW),
                             sp0.reshape(4 * NW, NP), skip0,
                             Wq1, Wk1, Wv1, Ws1)
    acc1, sp1 = _edge_sc(q1.reshape(NC * NP, GW), k1.reshape(NC * NP, GW),
                         v1.reshape(NC * NP, GW), src, dst)
    h2, s1, s2, cnt = _norm1(acc1.reshape(NC, NP, GW),
                             sp1.reshape(4 * NW, NP), skip1, batch2d)
    out = _norm2(h2, batch2d, s1, s2, cnt, gamma.reshape(1, D),
                 beta.reshape(1, D), mean_scale.reshape(1, D))
    return out[:N]


# speedup vs baseline: 1.0446x; 1.0446x over previous
import dataclasses
import functools

import jax
import jax.numpy as jnp
from jax import lax
from jax.experimental import pallas as pl
from jax.experimental.pallas import tpu as pltpu
from jax.experimental.pallas import tpu_sc as plsc

N = 10000
NP = 10240
D = 256
H = 8
G = 16
E = 160000
EP = 163840
DH = D // H
NC = 2
NT = 16
HPC = H // NC
GW = HPC * DH
EPT = EP // NT
C = 32
C2 = 512
NCHUNK = EPT // C
RPT = NP // NT
ZROWS = 16
SPN = HPC * NP
BR = 1024
NW = NC * NT
INV_SQRT_DH = 1.0 / (DH ** 0.5)


def _sel_mat():
    r = lax.broadcasted_iota(jnp.int32, (H, 4 * NW), 1)
    h = lax.broadcasted_iota(jnp.int32, (H, 4 * NW), 0)
    head = (r // (4 * NT)) * HPC + (r % HPC)
    return (head == h).astype(jnp.float32)


def _expand8_mat():
    r = lax.broadcasted_iota(jnp.int32, (H, D), 0)
    c = lax.broadcasted_iota(jnp.int32, (H, D), 1)
    return (r == c // DH).astype(jnp.float32)


def _agg_from_acc(acc_block, sp_block):
    s_heads = jnp.dot(_sel_mat(), sp_block,
                      preferred_element_type=jnp.float32)
    recip = 1.0 / (s_heads + 1e-16)
    dn = (((0,), (0,)), ((), ()))
    r_cols = lax.dot_general(recip, _expand8_mat(), dn,
                             preferred_element_type=jnp.float32)
    agg = jnp.concatenate([acc_block[0], acc_block[1]], axis=1)
    return agg * r_cols


def _proj_stores(h, wq_ref, wk_ref, wv_ref, ws_ref, q_ref, k_ref, v_ref, s_ref):
    for w_ref, o_ref in ((wq_ref, q_ref), (wk_ref, k_ref), (wv_ref, v_ref)):
        r = jnp.dot(h, w_ref[...], preferred_element_type=jnp.float32)
        o_ref[0] = r[:, :GW]
        o_ref[1] = r[:, GW:]
    s_ref[...] = jnp.dot(h, ws_ref[...], preferred_element_type=jnp.float32)


def _proj_body(x_ref, wq_ref, wk_ref, wv_ref, ws_ref, q_ref, k_ref, v_ref, s_ref):
    _proj_stores(x_ref[...], wq_ref, wk_ref, wv_ref, ws_ref,
                 q_ref, k_ref, v_ref, s_ref)


def _mid_body(acc_ref, sp_ref, skip_ref, wq_ref, wk_ref, wv_ref, ws_ref,
              q_ref, k_ref, v_ref, s_ref):
    h1 = _agg_from_acc(acc_ref[...], sp_ref[...]) + skip_ref[...]
    _proj_stores(h1, wq_ref, wk_ref, wv_ref, ws_ref, q_ref, k_ref, v_ref, s_ref)


def _project(x, Wq, Wk, Wv, Ws):
    wspec = pl.BlockSpec((D, D), lambda i: (0, 0))
    return pl.pallas_call(
        _proj_body,
        grid=(NP // BR,),
        in_specs=[pl.BlockSpec((BR, D), lambda i: (i, 0))] + [wspec] * 4,
        out_specs=[pl.BlockSpec((NC, BR, GW), lambda i: (0, i, 0))] * 3
        + [pl.BlockSpec((BR, D), lambda i: (i, 0))],
        out_shape=[jax.ShapeDtypeStruct((NC, NP, GW), jnp.float32)] * 3
        + [jax.ShapeDtypeStruct((NP, D), jnp.float32)],
    )(x, Wq, Wk, Wv, Ws)


def _mid(acc, sp, skip, Wq, Wk, Wv, Ws):
    wspec = pl.BlockSpec((D, D), lambda i: (0, 0))
    return pl.pallas_call(
        _mid_body,
        grid=(NP // BR,),
        in_specs=[pl.BlockSpec((NC, BR, GW), lambda i: (0, i, 0)),
                  pl.BlockSpec((4 * NW, BR), lambda i: (0, i)),
                  pl.BlockSpec((BR, D), lambda i: (i, 0))] + [wspec] * 4,
        out_specs=[pl.BlockSpec((NC, BR, GW), lambda i: (0, i, 0))] * 3
        + [pl.BlockSpec((BR, D), lambda i: (i, 0))],
        out_shape=[jax.ShapeDtypeStruct((NC, NP, GW), jnp.float32)] * 3
        + [jax.ShapeDtypeStruct((NP, D), jnp.float32)],
    )(acc, sp, skip, Wq, Wk, Wv, Ws)


def _edge_sc(qt, kt, vt, src, dst):
    mesh = plsc.VectorSubcoreMesh(core_axis_name="c", subcore_axis_name="s")
    cp = pltpu.CompilerParams()
    if "needs_layout_passes" in pltpu.CompilerParams.__dataclass_fields__:
        cp = dataclasses.replace(cp, needs_layout_passes=False)

    @functools.partial(
        pl.kernel,
        out_type=[jax.ShapeDtypeStruct((NC * NP, GW), jnp.float32),
                  jax.ShapeDtypeStruct((NC * EP, 16), jnp.float32)],
        mesh=mesh,
        compiler_params=cp,
        scratch_types=[
            pltpu.VMEM((C,), jnp.int32),
            pltpu.VMEM((C,), jnp.int32),
            pltpu.VMEM((C,), jnp.int32),
            pltpu.VMEM((C,), jnp.int32),
            pltpu.VMEM((C,), jnp.int32),
            pltpu.VMEM((C,), jnp.int32),
            pltpu.VMEM((C, GW), jnp.float32),
            pltpu.VMEM((C, GW), jnp.float32),
            pltpu.VMEM((C, GW), jnp.float32),
            pltpu.VMEM((C, GW), jnp.float32),
            pltpu.VMEM((C, GW), jnp.float32),
            pltpu.VMEM((C, GW), jnp.float32),
            pltpu.VMEM((C, 16), jnp.float32),
            pltpu.VMEM((C, 16), jnp.float32),
            pltpu.SemaphoreType.DMA,
            pltpu.SemaphoreType.DMA,
            pltpu.SemaphoreType.DMA,
            pltpu.SemaphoreType.DMA,
            pltpu.SemaphoreType.DMA,
            pltpu.SemaphoreType.DMA,
            pltpu.SemaphoreType.DMA,
            pltpu.SemaphoreType.DMA,
            pltpu.VMEM_SHARED((NP, GW), jnp.float32),
        ],
    )
    def k1(qt_h, kt_h, vt_h, srcg_h, dstg_h, zr_h, acc_h, w_h,
           sg0, sg1, dg0, dg1, dsc0, dsc1, qd0, qd1, kd0, kd1, vd0, vd1,
           wb0, wb1, isem0, isem1, gsem0, gsem1, ssem0, ssem1, wsem0, wsem1,
           accs):
        g = lax.axis_index("c")
        sid = lax.axis_index("s")
        goff = g * NP
        iota16 = lax.iota(jnp.int32, 16)
        ebase = sid * EPT

        sg = (sg0, sg1)
        dg = (dg0, dg1)
        dsc = (dsc0, dsc1)
        qd = (qd0, qd1)
        kd = (kd0, kd1)
        vd = (vd0, vd1)
        wb = (wb0, wb1)
        isem = (isem0, isem1)
        gsem = (gsem0, gsem1)
        ssem = (ssem0, ssem1)
        wsem = (wsem0, wsem1)

        pltpu.sync_copy(zr_h, accs.at[pl.ds(sid * RPT, RPT)])
        plsc.subcore_barrier()

        def idx_issue(j, s):
            base = g * EP + ebase + j * C
            pltpu.async_copy(srcg_h.at[pl.ds(base, C)], sg[s], isem[s])
            pltpu.async_copy(dstg_h.at[pl.ds(base, C)], dg[s], isem[s])

        def idx_wait(j, s):
            base = g * EP + ebase + j * C
            pltpu.make_async_copy(srcg_h.at[pl.ds(base, C)], sg[s],
                                  isem[s]).wait()
            pltpu.make_async_copy(dstg_h.at[pl.ds(base, C)], dg[s],
                                  isem[s]).wait()

        def gather_issue(s):
            pltpu.async_copy(qt_h.at[dg[s]], qd[s], gsem[s])
            pltpu.async_copy(kt_h.at[sg[s]], kd[s], gsem[s])
            pltpu.async_copy(vt_h.at[sg[s]], vd[s], gsem[s])

        def gather_wait(s):
            pltpu.make_async_copy(qt_h.at[dg[s]], qd[s], gsem[s]).wait()
            pltpu.make_async_copy(kt_h.at[sg[s]], kd[s], gsem[s]).wait()
            pltpu.make_async_copy(vt_h.at[sg[s]], vd[s], gsem[s]).wait()

        def out_issue(j, s):
            pltpu.async_copy(vd[s], accs.at[dsc[s]], ssem[s], add=True)
            pltpu.async_copy(wb[s], w_h.at[pl.ds(g * EP + ebase + j * C, C)],
                             wsem[s])

        def out_wait(j, s):
            pltpu.make_async_copy(vd[s], accs.at[dsc[s]], ssem[s]).wait()
            pltpu.make_async_copy(wb[s],
                                  w_h.at[pl.ds(g * EP + ebase + j * C, C)],
                                  wsem[s]).wait()

        def compute(s):
            for off in range(0, C, 16):
                sl = pl.ds(off, 16)
                dsc[s][sl] = dg[s][sl] - goff

            @pl.loop(0, C // 16)
            def _(grp):
                rows = iota16 + grp * 16
                for j in range(HPC):
                    cb = j * DH

                    def dbody(d4, acc4, cb=cb, rows=rows, s=s):
                        out = []
                        for u in range(4):
                            dd = d4 * 4 + u
                            cols = cb + ((dd + iota16) & (DH - 1))
                            qv = plsc.load_gather(qd[s], [rows, cols])
                            kv = plsc.load_gather(kd[s], [rows, cols])
                            out.append(acc4[u] + qv * kv)
                        return tuple(out)

                    z16 = jnp.zeros((16,), jnp.float32)
                    a4 = lax.fori_loop(0, DH // 4, dbody, (z16, z16, z16, z16))
                    sc = (a4[0] + a4[1]) + (a4[2] + a4[3])
                    w = jnp.exp(sc * INV_SQRT_DH)
                    plsc.store_scatter(
                        wb[s], [rows, (rows + j) & 15], w)

                    def d2body(dd, carry, cb=cb, rows=rows, w=w, s=s):
                        cols = cb + ((dd + iota16) & (DH - 1))
                        vg = plsc.load_gather(vd[s], [rows, cols])
                        plsc.store_scatter(vd[s], [rows, cols], vg * w)
                        return carry

                    lax.fori_loop(0, DH, d2body, 0, unroll=4)

        idx_issue(0, 0)
        idx_issue(1, 1)
        idx_wait(0, 0)
        gather_issue(0)

        gather_wait(0)
        compute(0)
        idx_wait(1, 1)
        gather_issue(1)
        idx_issue(2, 0)
        out_issue(0, 0)

        @pl.loop(0, (NCHUNK - 2) // 2)
        def _(cp2):
            for half in range(2):
                j = 1 + cp2 * 2 + half
                s = (1 + half) % 2
                o = 1 - s
                gather_wait(s)
                compute(s)
                idx_wait(j + 1, o)
                out_wait(j - 1, o)
                gather_issue(o)

                @pl.when(j < NCHUNK - 2)
                def _(j=j, s=s):
                    idx_issue(j + 2, s)

                out_issue(j, s)

        gather_wait(1)
        compute(1)
        out_wait(NCHUNK - 2, 0)
        out_issue(NCHUNK - 1, 1)
        out_wait(NCHUNK - 1, 1)

        plsc.subcore_barrier()

        @pl.loop(0, RPT // ZROWS)
        def _(b):
            r0 = sid * RPT + b * ZROWS
            pltpu.sync_copy(accs.at[pl.ds(r0, ZROWS)],
                            acc_h.at[pl.ds(goff + r0, ZROWS)])

    @functools.partial(
        pl.kernel,
        out_type=jax.ShapeDtypeStruct((NW, SPN), jnp.float32),
        mesh=mesh,
        compiler_params=cp,
        scratch_types=[
            pltpu.VMEM((C2,), jnp.int32),
            pltpu.VMEM((C2, 16), jnp.float32),
            pltpu.VMEM((SPN,), jnp.float32),
            pltpu.SemaphoreType.DMA,
        ],
    )
    def k2(w_h, dst_h, sp_h, dbuf, wb, spart, sem):
        g = lax.axis_index("c")
        sid = lax.axis_index("s")
        zvec = jnp.zeros((16,), jnp.float32)
        iota16 = lax.iota(jnp.int32, 16)
        jmod = (iota16 % HPC) * NP
        jmask = iota16 < HPC

        @pl.loop(0, SPN // 16)
        def _(r):
            spart[pl.ds(r * 16, 16)] = zvec

        ebase = sid * EPT

        @pl.loop(0, EPT // C2)
        def _(ci):
            base = ebase + ci * C2
            cd = pltpu.async_copy(dst_h.at[pl.ds(base, C2)], dbuf, sem)
            cw = pltpu.async_copy(w_h.at[pl.ds(g * EP + base, C2)], wb, sem)
            cd.wait()
            cw.wait()

            def ebody(eg, carry):
                e0 = eg * 16
                rows = iota16 + e0
                dvec = dbuf[pl.ds(e0, 16)]
                for j in range(HPC):
                    wj = plsc.load_gather(wb, [rows, (rows + j) & 15])
                    plsc.addupdate_scatter(spart, [j * NP + dvec], wj)
                return carry

            lax.fori_loop(0, C2 // 16, ebody, 0)

        pltpu.sync_copy(spart, sp_h.at[g * NT + sid])

    pad = jnp.full((EP - E,), N, jnp.int32)
    srcp = jnp.concatenate([src, pad])
    dstp = jnp.concatenate([dst, pad])
    srcg = jnp.concatenate([srcp, srcp + NP])
    dstg = jnp.concatenate([dstp, dstp + NP])
    zr = jnp.zeros((RPT, GW), jnp.float32)
    acc, w_all = k1(qt, kt, vt, srcg, dstg, zr)
    sp = k2(w_all, dstp)
    return acc, sp


def _norm1_body(acc_ref, sp_ref, skip_ref, b_ref, h2_ref, s1_ref, s2_ref,
                cnt_ref):
    i = pl.program_id(0)
    h2 = _agg_from_acc(acc_ref[...], sp_ref[...]) + skip_ref[...]
    h2_ref[...] = h2
    onehot = (b_ref[...] == lax.broadcasted_iota(jnp.int32, (1, G), 1))
    onehot = onehot.astype(jnp.float32)
    dn = (((0,), (0,)), ((), ()))
    p1 = lax.dot_general(onehot, h2, dn, preferred_element_type=jnp.float32)
    p2 = lax.dot_general(onehot, h2 * h2, dn,
                         preferred_element_type=jnp.float32)
    pc = lax.dot_general(onehot, jnp.ones_like(h2), dn,
                         preferred_element_type=jnp.float32)

    @pl.when(i == 0)
    def _():
        s1_ref[...] = jnp.zeros_like(s1_ref)
        s2_ref[...] = jnp.zeros_like(s2_ref)
        cnt_ref[...] = jnp.zeros_like(cnt_ref)

    s1_ref[...] += p1
    s2_ref[...] += p2
    cnt_ref[...] += pc


def _norm1(acc, sp, skip, batch2d):
    return pl.pallas_call(
        _norm1_body,
        grid=(NP // BR,),
        in_specs=[pl.BlockSpec((NC, BR, GW), lambda i: (0, i, 0)),
                  pl.BlockSpec((4 * NW, BR), lambda i: (0, i)),
                  pl.BlockSpec((BR, D), lambda i: (i, 0)),
                  pl.BlockSpec((BR, 1), lambda i: (i, 0))],
        out_specs=[pl.BlockSpec((BR, D), lambda i: (i, 0)),
                   pl.BlockSpec((G, D), lambda i: (0, 0)),
                   pl.BlockSpec((G, D), lambda i: (0, 0)),
                   pl.BlockSpec((G, D), lambda i: (0, 0))],
        out_shape=[jax.ShapeDtypeStruct((NP, D), jnp.float32),
                   jax.ShapeDtypeStruct((G, D), jnp.float32),
                   jax.ShapeDtypeStruct((G, D), jnp.float32),
                   jax.ShapeDtypeStruct((G, D), jnp.float32)],
    )(acc, sp, skip, batch2d)


def _norm2_body(h2_ref, b_ref, s1_ref, s2_ref, cnt_ref,
                gamma_ref, beta_ref, ms_ref, o_ref):
    cnt = jnp.maximum(cnt_ref[...], 1.0)
    mean = s1_ref[...] / cnt
    ex2 = s2_ref[...] / cnt
    ms = ms_ref[...]
    var = ex2 - (2.0 * ms - ms * ms) * (mean * mean)
    onehot = (b_ref[...] == lax.broadcasted_iota(jnp.int32, (1, G), 1))
    onehot = onehot.astype(jnp.float32)
    mean_r = jnp.dot(onehot, mean, preferred_element_type=jnp.float32)
    var_r = jnp.dot(onehot, var, preferred_element_type=jnp.float32)
    xc = h2_ref[...] - ms * mean_r
    o_ref[...] = (gamma_ref[...] * xc * lax.rsqrt(var_r + 1e-5)
                  + beta_ref[...])


def _norm2(h2, batch2d, s1, s2, cnt, gamma, beta, ms):
    small = pl.BlockSpec((G, D), lambda i: (0, 0))
    vec = pl.BlockSpec((1, D), lambda i: (0, 0))
    return pl.pallas_call(
        _norm2_body,
        grid=(NP // BR,),
        in_specs=[pl.BlockSpec((BR, D), lambda i: (i, 0)),
                  pl.BlockSpec((BR, 1), lambda i: (i, 0)),
                  small, small, small, vec, vec, vec],
        out_specs=pl.BlockSpec((BR, D), lambda i: (i, 0)),
        out_shape=jax.ShapeDtypeStruct((NP, D), jnp.float32),
    )(h2, batch2d, s1, s2, cnt, gamma, beta, ms)


def kernel(x, edge_index, batch, Wq0, Wk0, Wv0, Ws0,
           Wq1, Wk1, Wv1, Ws1, gamma, beta, mean_scale):
    src = edge_index[0]
    dst = edge_index[1]
    xp = jnp.pad(x, ((0, NP - N), (0, 0)))
    batch2d = jnp.pad(batch, (0, NP - N), constant_values=G).reshape(NP, 1)

    q0, k0, v0, skip0 = _project(xp, Wq0, Wk0, Wv0, Ws0)
    acc0, sp0 = _edge_sc(q0.reshape(NC * NP, GW), k0.reshape(NC * NP, GW),
                         v0.reshape(NC * NP, GW), src, dst)
    q1, k1, v1, skip1 = _mid(acc0.reshape(NC, NP, GW),
                             sp0.reshape(4 * NW, NP), skip0,
                             Wq1, Wk1, Wv1, Ws1)
    acc1, sp1 = _edge_sc(q1.reshape(NC * NP, GW), k1.reshape(NC * NP, GW),
                         v1.reshape(NC * NP, GW), src, dst)
    h2, s1, s2, cnt = _norm1(acc1.reshape(NC, NP, GW),
                             sp1.reshape(4 * NW, NP), skip1, batch2d)
    out = _norm2(h2, batch2d, s1, s2, cnt, gamma.reshape(1, D),
                 beta.reshape(1, D), mean_scale.reshape(1, D))
    return out[:N]


# speedup vs baseline: 1.4404x; 1.3788x over previous
import dataclasses
import functools

import jax
import jax.numpy as jnp
from jax import lax
from jax.experimental import pallas as pl
from jax.experimental.pallas import tpu as pltpu
from jax.experimental.pallas import tpu_sc as plsc

N = 10000
NP = 10240
D = 256
H = 8
G = 16
E = 160000
EP = 163840
DH = D // H
NC = 2
NT = 16
HPC = H // NC
GW = HPC * DH
EPT = EP // NT
C = 32
C2 = 512
NCHUNK = EPT // C
RPT = NP // NT
ZROWS = 16
SPN = HPC * NP
BR = 1024
NW = NC * NT
INV_SQRT_DH = 1.0 / (DH ** 0.5)


def _sel_mat():
    r = lax.broadcasted_iota(jnp.int32, (H, 4 * NW), 1)
    h = lax.broadcasted_iota(jnp.int32, (H, 4 * NW), 0)
    head = (r // (4 * NT)) * HPC + (r % HPC)
    return (head == h).astype(jnp.float32)


def _expand8_mat():
    r = lax.broadcasted_iota(jnp.int32, (H, D), 0)
    c = lax.broadcasted_iota(jnp.int32, (H, D), 1)
    return (r == c // DH).astype(jnp.float32)


def _agg_from_acc(acc_block, sp_block):
    s_heads = jnp.dot(_sel_mat(), sp_block,
                      preferred_element_type=jnp.float32)
    recip = 1.0 / (s_heads + 1e-16)
    dn = (((0,), (0,)), ((), ()))
    r_cols = lax.dot_general(recip, _expand8_mat(), dn,
                             preferred_element_type=jnp.float32)
    agg = jnp.concatenate([acc_block[0], acc_block[1]], axis=1)
    return agg * r_cols


def _proj_stores(h, wq_ref, wk_ref, wv_ref, ws_ref, q_ref, k_ref, v_ref, s_ref):
    for w_ref, o_ref in ((wq_ref, q_ref), (wk_ref, k_ref), (wv_ref, v_ref)):
        r = jnp.dot(h, w_ref[...], preferred_element_type=jnp.float32)
        o_ref[0] = r[:, :GW]
        o_ref[1] = r[:, GW:]
    s_ref[...] = jnp.dot(h, ws_ref[...], preferred_element_type=jnp.float32)


def _proj_body(x_ref, wq_ref, wk_ref, wv_ref, ws_ref, q_ref, k_ref, v_ref, s_ref):
    _proj_stores(x_ref[...], wq_ref, wk_ref, wv_ref, ws_ref,
                 q_ref, k_ref, v_ref, s_ref)


def _mid_body(acc_ref, sp_ref, skip_ref, wq_ref, wk_ref, wv_ref, ws_ref,
              q_ref, k_ref, v_ref, s_ref):
    h1 = _agg_from_acc(acc_ref[...], sp_ref[...]) + skip_ref[...]
    _proj_stores(h1, wq_ref, wk_ref, wv_ref, ws_ref, q_ref, k_ref, v_ref, s_ref)


def _project(x, Wq, Wk, Wv, Ws):
    wspec = pl.BlockSpec((D, D), lambda i: (0, 0))
    return pl.pallas_call(
        _proj_body,
        grid=(NP // BR,),
        in_specs=[pl.BlockSpec((BR, D), lambda i: (i, 0))] + [wspec] * 4,
        out_specs=[pl.BlockSpec((NC, BR, GW), lambda i: (0, i, 0))] * 3
        + [pl.BlockSpec((BR, D), lambda i: (i, 0))],
        out_shape=[jax.ShapeDtypeStruct((NC, NP, GW), jnp.float32)] * 3
        + [jax.ShapeDtypeStruct((NP, D), jnp.float32)],
    )(x, Wq, Wk, Wv, Ws)


def _mid(acc, sp, skip, Wq, Wk, Wv, Ws):
    wspec = pl.BlockSpec((D, D), lambda i: (0, 0))
    return pl.pallas_call(
        _mid_body,
        grid=(NP // BR,),
        in_specs=[pl.BlockSpec((NC, BR, GW), lambda i: (0, i, 0)),
                  pl.BlockSpec((4 * NW, BR), lambda i: (0, i)),
                  pl.BlockSpec((BR, D), lambda i: (i, 0))] + [wspec] * 4,
        out_specs=[pl.BlockSpec((NC, BR, GW), lambda i: (0, i, 0))] * 3
        + [pl.BlockSpec((BR, D), lambda i: (i, 0))],
        out_shape=[jax.ShapeDtypeStruct((NC, NP, GW), jnp.float32)] * 3
        + [jax.ShapeDtypeStruct((NP, D), jnp.float32)],
    )(acc, sp, skip, Wq, Wk, Wv, Ws)


def _edge_sc(qt, kt, vt, src, dst):
    mesh = plsc.VectorSubcoreMesh(core_axis_name="c", subcore_axis_name="s")
    cp = pltpu.CompilerParams()
    if "needs_layout_passes" in pltpu.CompilerParams.__dataclass_fields__:
        cp = dataclasses.replace(cp, needs_layout_passes=False)

    @functools.partial(
        pl.kernel,
        out_type=[jax.ShapeDtypeStruct((NC * NP, GW), jnp.float32),
                  jax.ShapeDtypeStruct((NC * EP, 16), jnp.float32)],
        mesh=mesh,
        compiler_params=cp,
        scratch_types=[
            pltpu.VMEM((C,), jnp.int32),
            pltpu.VMEM((C,), jnp.int32),
            pltpu.VMEM((C,), jnp.int32),
            pltpu.VMEM((C,), jnp.int32),
            pltpu.VMEM((C,), jnp.int32),
            pltpu.VMEM((C,), jnp.int32),
            pltpu.VMEM((C, GW), jnp.float32),
            pltpu.VMEM((C, GW), jnp.float32),
            pltpu.VMEM((C, GW), jnp.float32),
            pltpu.VMEM((C, GW), jnp.float32),
            pltpu.VMEM((C, GW), jnp.float32),
            pltpu.VMEM((C, GW), jnp.float32),
            pltpu.VMEM((C, 16), jnp.float32),
            pltpu.VMEM((C, 16), jnp.float32),
            pltpu.SemaphoreType.DMA,
            pltpu.SemaphoreType.DMA,
            pltpu.SemaphoreType.DMA,
            pltpu.SemaphoreType.DMA,
            pltpu.SemaphoreType.DMA,
            pltpu.SemaphoreType.DMA,
            pltpu.SemaphoreType.DMA,
            pltpu.SemaphoreType.DMA,
            pltpu.VMEM_SHARED((NP, GW), jnp.float32),
        ],
    )
    def k1(qt_h, kt_h, vt_h, srcg_h, dstg_h, zr_h, acc_h, w_h,
           sg0, sg1, dg0, dg1, dsc0, dsc1, qd0, qd1, kd0, kd1, vd0, vd1,
           wb0, wb1, isem0, isem1, gsem0, gsem1, ssem0, ssem1, wsem0, wsem1,
           accs):
        g = lax.axis_index("c")
        sid = lax.axis_index("s")
        goff = g * NP
        iota16 = lax.iota(jnp.int32, 16)
        ebase = sid * EPT

        sg = (sg0, sg1)
        dg = (dg0, dg1)
        dsc = (dsc0, dsc1)
        qd = (qd0, qd1)
        kd = (kd0, kd1)
        vd = (vd0, vd1)
        wb = (wb0, wb1)
        isem = (isem0, isem1)
        gsem = (gsem0, gsem1)
        ssem = (ssem0, ssem1)
        wsem = (wsem0, wsem1)

        pltpu.sync_copy(zr_h, accs.at[pl.ds(sid * RPT, RPT)])
        plsc.subcore_barrier()

        def idx_issue(j, s):
            base = g * EP + ebase + j * C
            pltpu.async_copy(srcg_h.at[pl.ds(base, C)], sg[s], isem[s])
            pltpu.async_copy(dstg_h.at[pl.ds(base, C)], dg[s], isem[s])

        def idx_wait(j, s):
            base = g * EP + ebase + j * C
            pltpu.make_async_copy(srcg_h.at[pl.ds(base, C)], sg[s],
                                  isem[s]).wait()
            pltpu.make_async_copy(dstg_h.at[pl.ds(base, C)], dg[s],
                                  isem[s]).wait()

        def gather_issue(s):
            pltpu.async_copy(qt_h.at[dg[s]], qd[s], gsem[s])
            pltpu.async_copy(kt_h.at[sg[s]], kd[s], gsem[s])
            pltpu.async_copy(vt_h.at[sg[s]], vd[s], gsem[s])

        def gather_wait(s):
            pltpu.make_async_copy(qt_h.at[dg[s]], qd[s], gsem[s]).wait()
            pltpu.make_async_copy(kt_h.at[sg[s]], kd[s], gsem[s]).wait()
            pltpu.make_async_copy(vt_h.at[sg[s]], vd[s], gsem[s]).wait()

        def out_issue(j, s):
            pltpu.async_copy(vd[s], accs.at[dsc[s]], ssem[s], add=True)
            pltpu.async_copy(wb[s], w_h.at[pl.ds(g * EP + ebase + j * C, C)],
                             wsem[s])

        def out_wait(j, s):
            pltpu.make_async_copy(vd[s], accs.at[dsc[s]], ssem[s]).wait()
            pltpu.make_async_copy(wb[s],
                                  w_h.at[pl.ds(g * EP + ebase + j * C, C)],
                                  wsem[s]).wait()

        def dsc_comp(s):
            for off in range(0, C, 16):
                sl = pl.ds(off, 16)
                dsc[s][sl] = dg[s][sl] - goff

        def compute(s):
            @pl.loop(0, C // 16)
            def _(grp):
                rows = iota16 + grp * 16
                for j in range(HPC):
                    cb = j * DH

                    def dbody(d4, acc4, cb=cb, rows=rows, s=s):
                        out = []
                        for u in range(4):
                            dd = d4 * 4 + u
                            cols = cb + ((dd + iota16) & (DH - 1))
                            qv = plsc.load_gather(qd[s], [rows, cols])
                            kv = plsc.load_gather(kd[s], [rows, cols])
                            out.append(acc4[u] + qv * kv)
                        return tuple(out)

                    z16 = jnp.zeros((16,), jnp.float32)
                    a4 = lax.fori_loop(0, DH // 4, dbody, (z16, z16, z16, z16))
                    sc = (a4[0] + a4[1]) + (a4[2] + a4[3])
                    w = jnp.exp(sc * INV_SQRT_DH)
                    plsc.store_scatter(
                        wb[s], [rows, (rows + j) & 15], w)

                    def d2body(dd, carry, cb=cb, rows=rows, w=w, s=s):
                        cols = cb + ((dd + iota16) & (DH - 1))
                        vg = plsc.load_gather(vd[s], [rows, cols])
                        plsc.store_scatter(vd[s], [rows, cols], vg * w)
                        return carry

                    lax.fori_loop(0, DH, d2body, 0, unroll=4)

        idx_issue(0, 0)
        idx_issue(1, 1)
        idx_wait(0, 0)
        gather_issue(0)

        gather_wait(0)
        idx_wait(1, 1)
        gather_issue(1)
        dsc_comp(0)
        idx_issue(2, 0)
        compute(0)
        out_issue(0, 0)

        @pl.loop(0, (NCHUNK - 2) // 2)
        def _(cp2):
            for half in range(2):
                j = 1 + cp2 * 2 + half
                s = (1 + half) % 2
                o = 1 - s
                gather_wait(s)
                idx_wait(j + 1, o)
                out_wait(j - 1, o)
                gather_issue(o)
                dsc_comp(s)

                @pl.when(j < NCHUNK - 2)
                def _(j=j, s=s):
                    idx_issue(j + 2, s)

                compute(s)
                out_issue(j, s)

        gather_wait(1)
        out_wait(NCHUNK - 2, 0)
        dsc_comp(1)
        compute(1)
        out_issue(NCHUNK - 1, 1)
        out_wait(NCHUNK - 1, 1)

        plsc.subcore_barrier()

        @pl.loop(0, RPT // ZROWS)
        def _(b):
            r0 = sid * RPT + b * ZROWS
            pltpu.sync_copy(accs.at[pl.ds(r0, ZROWS)],
                            acc_h.at[pl.ds(goff + r0, ZROWS)])

    @functools.partial(
        pl.kernel,
        out_type=jax.ShapeDtypeStruct((NW, SPN), jnp.float32),
        mesh=mesh,
        compiler_params=cp,
        scratch_types=[
            pltpu.VMEM((C2,), jnp.int32),
            pltpu.VMEM((C2, 16), jnp.float32),
            pltpu.VMEM((SPN,), jnp.float32),
            pltpu.SemaphoreType.DMA,
        ],
    )
    def k2(w_h, dst_h, sp_h, dbuf, wb, spart, sem):
        g = lax.axis_index("c")
        sid = lax.axis_index("s")
        zvec = jnp.zeros((16,), jnp.float32)
        iota16 = lax.iota(jnp.int32, 16)
        jmod = (iota16 % HPC) * NP
        jmask = iota16 < HPC

        @pl.loop(0, SPN // 16)
        def _(r):
            spart[pl.ds(r * 16, 16)] = zvec

        ebase = sid * EPT

        @pl.loop(0, EPT // C2)
        def _(ci):
            base = ebase + ci * C2
            cd = pltpu.async_copy(dst_h.at[pl.ds(base, C2)], dbuf, sem)
            cw = pltpu.async_copy(w_h.at[pl.ds(g * EP + base, C2)], wb, sem)
            cd.wait()
            cw.wait()

            def ebody(eg, carry):
                e0 = eg * 16
                rows = iota16 + e0
                dvec = dbuf[pl.ds(e0, 16)]
                for j in range(HPC):
                    wj = plsc.load_gather(wb, [rows, (rows + j) & 15])
                    plsc.addupdate_scatter(spart, [j * NP + dvec], wj)
                return carry

            lax.fori_loop(0, C2 // 16, ebody, 0)

        pltpu.sync_copy(spart, sp_h.at[g * NT + sid])

    pad = jnp.full((EP - E,), N, jnp.int32)
    srcp = jnp.concatenate([src, pad])
    dstp = jnp.concatenate([dst, pad])
    srcg = jnp.concatenate([srcp, srcp + NP])
    dstg = jnp.concatenate([dstp, dstp + NP])
    zr = jnp.zeros((RPT, GW), jnp.float32)
    acc, w_all = k1(qt, kt, vt, srcg, dstg, zr)
    sp = k2(w_all, dstp)
    return acc, sp


def _norm1_body(acc_ref, sp_ref, skip_ref, b_ref, h2_ref, s1_ref, s2_ref,
                cnt_ref):
    i = pl.program_id(0)
    h2 = _agg_from_acc(acc_ref[...], sp_ref[...]) + skip_ref[...]
    h2_ref[...] = h2
    onehot = (b_ref[...] == lax.broadcasted_iota(jnp.int32, (1, G), 1))
    onehot = onehot.astype(jnp.float32)
    dn = (((0,), (0,)), ((), ()))
    p1 = lax.dot_general(onehot, h2, dn, preferred_element_type=jnp.float32)
    p2 = lax.dot_general(onehot, h2 * h2, dn,
                         preferred_element_type=jnp.float32)
    pc = lax.dot_general(onehot, jnp.ones_like(h2), dn,
                         preferred_element_type=jnp.float32)

    @pl.when(i == 0)
    def _():
        s1_ref[...] = jnp.zeros_like(s1_ref)
        s2_ref[...] = jnp.zeros_like(s2_ref)
        cnt_ref[...] = jnp.zeros_like(cnt_ref)

    s1_ref[...] += p1
    s2_ref[...] += p2
    cnt_ref[...] += pc


def _norm1(acc, sp, skip, batch2d):
    return pl.pallas_call(
        _norm1_body,
        grid=(NP // BR,),
        in_specs=[pl.BlockSpec((NC, BR, GW), lambda i: (0, i, 0)),
                  pl.BlockSpec((4 * NW, BR), lambda i: (0, i)),
                  pl.BlockSpec((BR, D), lambda i: (i, 0)),
                  pl.BlockSpec((BR, 1), lambda i: (i, 0))],
        out_specs=[pl.BlockSpec((BR, D), lambda i: (i, 0)),
                   pl.BlockSpec((G, D), lambda i: (0, 0)),
                   pl.BlockSpec((G, D), lambda i: (0, 0)),
                   pl.BlockSpec((G, D), lambda i: (0, 0))],
        out_shape=[jax.ShapeDtypeStruct((NP, D), jnp.float32),
                   jax.ShapeDtypeStruct((G, D), jnp.float32),
                   jax.ShapeDtypeStruct((G, D), jnp.float32),
                   jax.ShapeDtypeStruct((G, D), jnp.float32)],
    )(acc, sp, skip, batch2d)


def _norm2_body(h2_ref, b_ref, s1_ref, s2_ref, cnt_ref,
                gamma_ref, beta_ref, ms_ref, o_ref):
    cnt = jnp.maximum(cnt_ref[...], 1.0)
    mean = s1_ref[...] / cnt
    ex2 = s2_ref[...] / cnt
    ms = ms_ref[...]
    var = ex2 - (2.0 * ms - ms * ms) * (mean * mean)
    onehot = (b_ref[...] == lax.broadcasted_iota(jnp.int32, (1, G), 1))
    onehot = onehot.astype(jnp.float32)
    mean_r = jnp.dot(onehot, mean, preferred_element_type=jnp.float32)
    var_r = jnp.dot(onehot, var, preferred_element_type=jnp.float32)
    xc = h2_ref[...] - ms * mean_r
    o_ref[...] = (gamma_ref[...] * xc * lax.rsqrt(var_r + 1e-5)
                  + beta_ref[...])


def _norm2(h2, batch2d, s1, s2, cnt, gamma, beta, ms):
    small = pl.BlockSpec((G, D), lambda i: (0, 0))
    vec = pl.BlockSpec((1, D), lambda i: (0, 0))
    return pl.pallas_call(
        _norm2_body,
        grid=(NP // BR,),
        in_specs=[pl.BlockSpec((BR, D), lambda i: (i, 0)),
                  pl.BlockSpec((BR, 1), lambda i: (i, 0)),
                  small, small, small, vec, vec, vec],
        out_specs=pl.BlockSpec((BR, D), lambda i: (i, 0)),
        out_shape=jax.ShapeDtypeStruct((NP, D), jnp.float32),
    )(h2, batch2d, s1, s2, cnt, gamma, beta, ms)


def kernel(x, edge_index, batch, Wq0, Wk0, Wv0, Ws0,
           Wq1, Wk1, Wv1, Ws1, gamma, beta, mean_scale):
    src = edge_index[0]
    dst = edge_index[1]
    xp = jnp.pad(x, ((0, NP - N), (0, 0)))
    batch2d = jnp.pad(batch, (0, NP - N), constant_values=G).reshape(NP, 1)

    q0, k0, v0, skip0 = _project(xp, Wq0, Wk0, Wv0, Ws0)
    acc0, sp0 = _edge_sc(q0.reshape(NC * NP, GW), k0.reshape(NC * NP, GW),
                         v0.reshape(NC * NP, GW), src, dst)
    q1, k1, v1, skip1 = _mid(acc0.reshape(NC, NP, GW),
                             sp0.reshape(4 * NW, NP), skip0,
                             Wq1, Wk1, Wv1, Ws1)
    acc1, sp1 = _edge_sc(q1.reshape(NC * NP, GW), k1.reshape(NC * NP, GW),
                         v1.reshape(NC * NP, GW), src, dst)
    h2, s1, s2, cnt = _norm1(acc1.reshape(NC, NP, GW),
                             sp1.reshape(4 * NW, NP), skip1, batch2d)
    out = _norm2(h2, batch2d, s1, s2, cnt, gamma.reshape(1, D),
                 beta.reshape(1, D), mean_scale.reshape(1, D))
    return out[:N]
